# baseline (device time: 613344 ns/iter reference)
import jax
import jax.numpy as jnp
from jax import lax
from jax.experimental import pallas as pl
from jax.experimental.pallas import tpu as pltpu

NCHUNK = 8


def kernel(x):
    m, n2 = x.shape
    n = n2 // 2
    out_m = 2 * m
    x = jnp.transpose(x.astype(jnp.bfloat16).reshape(m, 2, n), (1, 0, 2))

    def body(x_ref, out_ref, send_sems, recv_sems, local_sem):
        my_x = lax.axis_index("x")
        my_y = lax.axis_index("y")
        my_z = lax.axis_index("z")
        partner_y = 1 - my_y
        partner = (my_x, partner_y, my_z)

        barrier_sem = pltpu.get_barrier_semaphore()
        pl.semaphore_signal(
            barrier_sem, inc=1,
            device_id=partner, device_id_type=pl.DeviceIdType.MESH,
        )
        pl.semaphore_wait(barrier_sem, 1)

        rows = m // NCHUNK
        rdmas = []
        for c in range(NCHUNK):
            rdma = pltpu.make_async_remote_copy(
                src_ref=x_ref.at[partner_y, pl.ds(c * rows, rows), :],
                dst_ref=out_ref.at[pl.ds(my_y * m + c * rows, rows), :],
                send_sem=send_sems.at[c],
                recv_sem=recv_sems.at[c],
                device_id=partner,
                device_id_type=pl.DeviceIdType.MESH,
            )
            rdma.start()
            rdmas.append(rdma)

        local = pltpu.make_async_copy(
            x_ref.at[my_y],
            out_ref.at[pl.ds(my_y * m, m), :],
            local_sem,
        )
        local.start()
        local.wait()
        for rdma in rdmas:
            rdma.wait()

    return pl.pallas_call(
        body,
        out_shape=jax.ShapeDtypeStruct((out_m, n), jnp.bfloat16),
        in_specs=[pl.BlockSpec(memory_space=pl.ANY)],
        out_specs=pl.BlockSpec(memory_space=pl.ANY),
        scratch_shapes=[
            pltpu.SemaphoreType.DMA((NCHUNK,)),
            pltpu.SemaphoreType.DMA((NCHUNK,)),
            pltpu.SemaphoreType.DMA,
        ],
        compiler_params=pltpu.CompilerParams(collective_id=0),
    )(x)


# device time: 207794 ns/iter; 2.9517x vs baseline; 2.9517x over previous
import jax
import jax.numpy as jnp
from jax import lax
from jax.experimental import pallas as pl
from jax.experimental.pallas import tpu as pltpu

NCHUNK = 16


def kernel(x):
    m, n2 = x.shape
    n = n2 // 2
    out_m = 2 * m
    rows = m // NCHUNK

    def body(x_ref, out_ref, vin, vsend, vlocal,
             in_sems, send_sems, recv_sems, local_sems):
        my_x = lax.axis_index("x")
        my_y = lax.axis_index("y")
        my_z = lax.axis_index("z")
        partner_y = 1 - my_y
        partner = (my_x, partner_y, my_z)

        barrier_sem = pltpu.get_barrier_semaphore()
        pl.semaphore_signal(
            barrier_sem, inc=1,
            device_id=partner, device_id_type=pl.DeviceIdType.MESH,
        )
        pl.semaphore_wait(barrier_sem, 1)

        def in_copy(c, slot):
            return pltpu.make_async_copy(
                x_ref.at[pl.ds(c * rows, rows), :], vin.at[slot],
                in_sems.at[slot],
            )

        def local_copy(c, slot):
            return pltpu.make_async_copy(
                vlocal.at[slot],
                out_ref.at[pl.ds(my_y * m + c * rows, rows), :],
                local_sems.at[slot],
            )

        in_copy(0, 0).start()
        for c in range(NCHUNK):
            slot = c % 2
            if c + 1 < NCHUNK:
                in_copy(c + 1, 1 - slot).start()
            in_copy(c, slot).wait()

            if c >= 2:
                local_copy(c - 2, slot).wait()

            @pl.when(my_y == 0)
            def _():
                chunk = vin[slot]
                vsend[pl.ds(c * rows, rows), :] = chunk[:, n:].astype(jnp.bfloat16)
                vlocal[slot] = chunk[:, :n].astype(jnp.bfloat16)

            @pl.when(my_y == 1)
            def _():
                chunk = vin[slot]
                vsend[pl.ds(c * rows, rows), :] = chunk[:, :n].astype(jnp.bfloat16)
                vlocal[slot] = chunk[:, n:].astype(jnp.bfloat16)

            rdma = pltpu.make_async_remote_copy(
                src_ref=vsend.at[pl.ds(c * rows, rows), :],
                dst_ref=out_ref.at[pl.ds(my_y * m + c * rows, rows), :],
                send_sem=send_sems.at[c],
                recv_sem=recv_sems.at[c],
                device_id=partner,
                device_id_type=pl.DeviceIdType.MESH,
            )
            rdma.start()
            local_copy(c, slot).start()

        for c in (NCHUNK - 2, NCHUNK - 1):
            local_copy(c, c % 2).wait()
        for c in range(NCHUNK):
            pltpu.make_async_copy(vsend.at[pl.ds(c * rows, rows), :],
                                  vsend.at[pl.ds(c * rows, rows), :],
                                  send_sems.at[c]).wait()
            pltpu.make_async_copy(out_ref.at[pl.ds(my_y * m + c * rows, rows), :],
                                  out_ref.at[pl.ds(my_y * m + c * rows, rows), :],
                                  recv_sems.at[c]).wait()

    return pl.pallas_call(
        body,
        out_shape=jax.ShapeDtypeStruct((out_m, n), jnp.bfloat16),
        in_specs=[pl.BlockSpec(memory_space=pl.ANY)],
        out_specs=pl.BlockSpec(memory_space=pl.ANY),
        scratch_shapes=[
            pltpu.VMEM((2, rows, n2), jnp.float32),
            pltpu.VMEM((m, n), jnp.bfloat16),
            pltpu.VMEM((2, rows, n), jnp.bfloat16),
            pltpu.SemaphoreType.DMA((2,)),
            pltpu.SemaphoreType.DMA((NCHUNK,)),
            pltpu.SemaphoreType.DMA((NCHUNK,)),
            pltpu.SemaphoreType.DMA((2,)),
        ],
        compiler_params=pltpu.CompilerParams(collective_id=0),
    )(x)
